# baseline (speedup 1.0000x reference)
import jax
import jax.numpy as jnp
from jax import lax
from jax.experimental import pallas as pl
from jax.experimental.pallas import tpu as pltpu
from jax.experimental.pallas import tpu_sc as plsc

_BATCH = 4096
_SEQ = 50
_D = 64
_DP = 128
_V = 1000000
_H = 507904
_B = _BATCH * _SEQ
_NC, _NS = 2, 16
_NW = _NC * _NS
_BPW = _B // _NW
_CHUNK = 400
_NCHUNK = _BPW // _CHUNK

_TBLK = 16384


def _tp_body(lo_ref, hi_ref, out_ref):
    out_ref[:, 0:_D] = lo_ref[...].T
    out_ref[:, _D:_DP] = hi_ref[...].T


def _emb_body(x_hbm, table_hbm, out_hbm, idx_v, pidx_v, rows0, rows1,
              gsem0, gsem1, wsem0, wsem1):
    wid = lax.axis_index("s") * _NC + lax.axis_index("c")
    base = wid * _BPW
    bufs = (rows0, rows1)
    gsems = (gsem0, gsem1)
    wsems = (wsem0, wsem1)

    pltpu.sync_copy(x_hbm.at[pl.ds(base, _BPW)], idx_v)

    def mkpair(i, carry):
        v = idx_v[pl.ds(i * 16, 16)]
        pidx_v[pl.ds(i * 16, 16)] = jnp.where(v >= _H, v - _H, v)
        return carry

    lax.fori_loop(0, _BPW // 16, mkpair, 0)

    def gather(g):
        return pltpu.async_copy(
            table_hbm.at[pidx_v.at[pl.ds(g * _CHUNK, _CHUNK)]],
            bufs[g % 2], gsems[g % 2])

    def writeback(g):
        return pltpu.async_copy(
            bufs[g % 2],
            out_hbm.at[pl.ds(base + g * _CHUNK, _CHUNK)],
            wsems[g % 2])

    g0 = gather(0)
    pending_g = [g0, None]
    pending_w = [None, None]
    for g in range(_NCHUNK):
        nxt = g + 1
        if nxt < _NCHUNK:
            if pending_w[nxt % 2] is not None:
                pending_w[nxt % 2].wait()
            pending_g[nxt % 2] = gather(nxt)
        pending_g[g % 2].wait()
        pending_w[g % 2] = writeback(g)
    pending_w[(_NCHUNK - 1) % 2].wait()
    pending_w[_NCHUNK % 2].wait()


def kernel(x, table):
    xf = x.reshape(_B)
    tt = table.T
    grid = _H // _TBLK
    t2 = pl.pallas_call(
        _tp_body,
        grid=(grid,),
        in_specs=[
            pl.BlockSpec((_D, _TBLK), lambda j: (0, j)),
            pl.BlockSpec((_D, _TBLK), lambda j: (0, j + _H // _TBLK)),
        ],
        out_specs=pl.BlockSpec((_TBLK, _DP), lambda j: (j, 0)),
        out_shape=jax.ShapeDtypeStruct((_H, _DP), jnp.float32),
    )(tt, tt)
    mesh = plsc.VectorSubcoreMesh(core_axis_name="c", subcore_axis_name="s")
    out = pl.kernel(
        _emb_body,
        out_type=jax.ShapeDtypeStruct((_B, _DP), jnp.float32),
        mesh=mesh,
        scratch_types=[
            pltpu.VMEM((_BPW,), jnp.int32),
            pltpu.VMEM((_BPW,), jnp.int32),
            pltpu.VMEM((_CHUNK, _DP), jnp.float32),
            pltpu.VMEM((_CHUNK, _DP), jnp.float32),
            pltpu.SemaphoreType.DMA,
            pltpu.SemaphoreType.DMA,
            pltpu.SemaphoreType.DMA,
            pltpu.SemaphoreType.DMA,
        ],
        compiler_params=pltpu.CompilerParams(use_tc_tiling_on_sc=True),
    )(xf, t2)
    hi = (xf >= _H)[:, None]
    sel = jnp.where(hi, out[:, _D:_DP], out[:, 0:_D])
    return sel.reshape(_BATCH, _SEQ, _D)


# speedup vs baseline: 1.2196x; 1.2196x over previous
import jax
import jax.numpy as jnp
from jax import lax
from jax.experimental import pallas as pl
from jax.experimental.pallas import tpu as pltpu
from jax.experimental.pallas import tpu_sc as plsc

_BATCH = 4096
_SEQ = 50
_D = 64
_DP = 128
_V = 1000000
_B = _BATCH * _SEQ
_NC, _NS = 2, 16
_NW = _NC * _NS
_BPW = _B // _NW
_CHUNK = 400
_NCHUNK = _BPW // _CHUNK

_TBLK = 32768


def _tp_body(in_ref, out_ref):
    out_ref[:, 0:_D] = in_ref[...].T


def _emb_body(x_hbm, table_hbm, out_hbm, idx_v, rows0, rows1, gsem0, gsem1,
              wsem0, wsem1):
    wid = lax.axis_index("s") * _NC + lax.axis_index("c")
    base = wid * _BPW
    bufs = (rows0, rows1)
    gsems = (gsem0, gsem1)
    wsems = (wsem0, wsem1)

    pltpu.sync_copy(x_hbm.at[pl.ds(base, _BPW)], idx_v)

    def gather(g):
        return pltpu.async_copy(
            table_hbm.at[idx_v.at[pl.ds(g * _CHUNK, _CHUNK)]],
            bufs[g % 2], gsems[g % 2])

    def writeback(g):
        return pltpu.async_copy(
            bufs[g % 2],
            out_hbm.at[pl.ds(base + g * _CHUNK, _CHUNK)],
            wsems[g % 2])

    g0 = gather(0)
    pending_g = [g0, None]
    pending_w = [None, None]
    for g in range(_NCHUNK):
        nxt = g + 1
        if nxt < _NCHUNK:
            if pending_w[nxt % 2] is not None:
                pending_w[nxt % 2].wait()
            pending_g[nxt % 2] = gather(nxt)
        pending_g[g % 2].wait()
        pending_w[g % 2] = writeback(g)
    pending_w[(_NCHUNK - 1) % 2].wait()
    pending_w[_NCHUNK % 2].wait()


def kernel(x, table):
    xf = x.reshape(_B)
    grid = pl.cdiv(_V, _TBLK)
    t2 = pl.pallas_call(
        _tp_body,
        grid=(grid,),
        in_specs=[pl.BlockSpec((_D, _TBLK), lambda j: (0, j))],
        out_specs=pl.BlockSpec((_TBLK, _DP), lambda j: (j, 0)),
        out_shape=jax.ShapeDtypeStruct((_V, _DP), jnp.float32),
    )(table.T)
    mesh = plsc.VectorSubcoreMesh(core_axis_name="c", subcore_axis_name="s")
    out = pl.kernel(
        _emb_body,
        out_type=jax.ShapeDtypeStruct((_B, _DP), jnp.float32),
        mesh=mesh,
        scratch_types=[
            pltpu.VMEM((_BPW,), jnp.int32),
            pltpu.VMEM((_CHUNK, _DP), jnp.float32),
            pltpu.VMEM((_CHUNK, _DP), jnp.float32),
            pltpu.SemaphoreType.DMA,
            pltpu.SemaphoreType.DMA,
            pltpu.SemaphoreType.DMA,
            pltpu.SemaphoreType.DMA,
        ],
        compiler_params=pltpu.CompilerParams(use_tc_tiling_on_sc=True),
    )(xf, t2)
    return out.reshape(_BATCH, _SEQ, _DP)[:, :, :_D]


# speedup vs baseline: 1.2211x; 1.0012x over previous
import jax
import jax.numpy as jnp
from jax import lax
from jax.experimental import pallas as pl
from jax.experimental.pallas import tpu as pltpu
from jax.experimental.pallas import tpu_sc as plsc

_BATCH = 4096
_SEQ = 50
_D = 64
_DP = 128
_V = 1000000
_B = _BATCH * _SEQ
_NC, _NS = 2, 16
_NW = _NC * _NS
_BPW = _B // _NW
_CHUNK = 400
_NCHUNK = _BPW // _CHUNK

_TBLK = 32768


def _tp_body(in_ref, out_ref):
    out_ref[:, 0:_D] = in_ref[...].T


def _emb_body(x_hbm, table_hbm, out_hbm, idx_v, rows0, rows1, gsem0, gsem1,
              wsem0, wsem1):
    wid = lax.axis_index("s") * _NC + lax.axis_index("c")
    base = wid * _BPW
    bufs = (rows0, rows1)
    gsems = (gsem0, gsem1)
    wsems = (wsem0, wsem1)

    pltpu.sync_copy(x_hbm.at[pl.ds(base, _BPW)], idx_v)

    def gather(g):
        return pltpu.async_copy(
            table_hbm.at[idx_v.at[pl.ds(g * _CHUNK, _CHUNK)]],
            bufs[g % 2], gsems[g % 2])

    def writeback(g):
        return pltpu.async_copy(
            bufs[g % 2],
            out_hbm.at[pl.ds(base + g * _CHUNK, _CHUNK)],
            wsems[g % 2])

    g0 = gather(0)
    pending_g = [g0, None]
    pending_w = [None, None]
    for g in range(_NCHUNK):
        nxt = g + 1
        if nxt < _NCHUNK:
            if pending_w[nxt % 2] is not None:
                pending_w[nxt % 2].wait()
            pending_g[nxt % 2] = gather(nxt)
        pending_g[g % 2].wait()
        pending_w[g % 2] = writeback(g)
    pending_w[(_NCHUNK - 1) % 2].wait()
    pending_w[_NCHUNK % 2].wait()


def kernel(x, table):
    xf = x.reshape(_B)
    grid = pl.cdiv(_V, _TBLK)
    t2 = pl.pallas_call(
        _tp_body,
        grid=(grid,),
        in_specs=[pl.BlockSpec((_D, _TBLK), lambda j: (0, j))],
        out_specs=pl.BlockSpec((_TBLK, _DP), lambda j: (j, 0)),
        out_shape=jax.ShapeDtypeStruct((_V, _DP), jnp.float32),
    )(table.T)
    mesh = plsc.VectorSubcoreMesh(core_axis_name="c", subcore_axis_name="s")
    out = pl.kernel(
        _emb_body,
        out_type=jax.ShapeDtypeStruct((_B, _DP), jnp.float32),
        mesh=mesh,
        scratch_types=[
            pltpu.VMEM((_BPW,), jnp.int32),
            pltpu.VMEM((_CHUNK, _DP), jnp.float32),
            pltpu.VMEM((_CHUNK, _DP), jnp.float32),
            pltpu.SemaphoreType.DMA,
            pltpu.SemaphoreType.DMA,
            pltpu.SemaphoreType.DMA,
            pltpu.SemaphoreType.DMA,
        ],
        compiler_params=pltpu.CompilerParams(use_tc_tiling_on_sc=True),
    )(xf, t2)
    return out[:, :_D].reshape(_BATCH, _SEQ, _D)
